# Initial kernel scaffold; baseline (speedup 1.0000x reference)
import jax
import jax.numpy as jnp
from jax.experimental import pallas as pl


def kernel(x, edge_index, batch, W1, b1, W2, b2, gamma, beta):
    raise NotImplementedError("write your pallas kernel here")


# speedup vs baseline: 2.9145x; 2.9145x over previous
import functools

import jax
import jax.numpy as jnp
from jax import lax
from jax.experimental import pallas as pl
from jax.experimental.pallas import tpu as pltpu
from jax.experimental.pallas import tpu_sc as plsc

N_NODES = 10000
N_EDGES = 320000
HID = 128
N_LAYERS = 3

NC = 2
NS = 16
NW = NC * NS

CHUNK = 128
NCHUNKS = 80
EPT = CHUNK * NCHUNKS
E_PAD = NW * EPT

DUMMY_ROW = N_NODES
AGG_ROWS = 10112
ZROWS_PER_TILE = AGG_ROWS // NS
OUT_ROWS_PER_TILE = 624

_sc_mesh = plsc.VectorSubcoreMesh(core_axis_name="c", subcore_axis_name="s")


@functools.partial(
    pl.kernel,
    out_type=jax.ShapeDtypeStruct((NC, N_NODES, HID), jnp.float32),
    mesh=_sc_mesh,
    scratch_types=[
        pltpu.VMEM((EPT,), jnp.int32),
        pltpu.VMEM((NCHUNKS, CHUNK), jnp.int32),
        pltpu.VMEM((CHUNK, HID), jnp.float32),
        pltpu.VMEM_SHARED((AGG_ROWS, HID), jnp.float32),
        pltpu.SemaphoreType.DMA,
    ],
)
def _sc_aggregate(h_hbm, src_hbm, dst_hbm, out_hbm,
                  src_v, dst_v, rows_v, agg_sh, sem):
    c = lax.axis_index("c")
    s = lax.axis_index("s")
    wid = c * NS + s

    pltpu.sync_copy(src_hbm.at[pl.ds(wid * EPT, EPT)], src_v)
    pltpu.sync_copy(dst_hbm.at[wid], dst_v)

    zeros16 = jnp.zeros((16,), jnp.float32)

    @pl.loop(0, CHUNK)
    def _(r):
        for cc in range(HID // 16):
            rows_v[r, pl.ds(cc * 16, 16)] = zeros16

    zbase = pl.multiple_of(s * ZROWS_PER_TILE, 8)
    for k in range(ZROWS_PER_TILE // CHUNK):
        pltpu.sync_copy(rows_v, agg_sh.at[pl.ds(zbase + k * CHUNK, CHUNK)])
    rem = ZROWS_PER_TILE % CHUNK
    if rem:
        pltpu.sync_copy(rows_v.at[pl.ds(0, rem)],
                        agg_sh.at[pl.ds(zbase + ZROWS_PER_TILE - rem, rem)])

    plsc.subcore_barrier()

    @pl.loop(0, NCHUNKS)
    def _(j):
        pltpu.async_copy(
            h_hbm.at[src_v.at[pl.ds(j * CHUNK, CHUNK)]], rows_v, sem
        ).wait()
        pltpu.sync_copy(rows_v, agg_sh.at[dst_v.at[j]], add=True)

    plsc.subcore_barrier()

    obase = pl.multiple_of(s * OUT_ROWS_PER_TILE, 8)

    @pl.when(s < NS - 1)
    def _():
        pltpu.sync_copy(agg_sh.at[pl.ds(obase, OUT_ROWS_PER_TILE)],
                        out_hbm.at[c].at[pl.ds(obase, OUT_ROWS_PER_TILE)])

    last_base = (NS - 1) * OUT_ROWS_PER_TILE
    last_rows = N_NODES - last_base

    @pl.when(s == NS - 1)
    def _():
        pltpu.sync_copy(agg_sh.at[pl.ds(last_base, last_rows)],
                        out_hbm.at[c].at[pl.ds(last_base, last_rows)])


def _mlp_body(h_ref, p_ref, w1_ref, b1_ref, w2_ref, b2_ref, g_ref, be_ref,
              o_ref):
    z = h_ref[...] + p_ref[0] + p_ref[1]
    z = jnp.dot(z, w1_ref[...], preferred_element_type=jnp.float32) + b1_ref[...]
    z = jnp.maximum(z, 0.0)
    z = jnp.dot(z, w2_ref[...], preferred_element_type=jnp.float32) + b2_ref[...]
    z = jnp.maximum(z, 0.0)
    mu = jnp.mean(z, axis=0, keepdims=True)
    zc = z - mu
    var = jnp.mean(zc * zc, axis=0, keepdims=True)
    o_ref[...] = zc * lax.rsqrt(var + 1e-5) * g_ref[...] + be_ref[...]


_mlp_call = pl.pallas_call(
    _mlp_body,
    out_shape=jax.ShapeDtypeStruct((N_NODES, HID), jnp.float32),
)


def kernel(x, edge_index, batch, W1, b1, W2, b2, gamma, beta):
    src = edge_index[0].astype(jnp.int32)
    dst = edge_index[1].astype(jnp.int32)
    pad = E_PAD - N_EDGES
    src_p = jnp.concatenate([src, jnp.zeros((pad,), jnp.int32)])
    dst_p = jnp.concatenate([dst, jnp.full((pad,), DUMMY_ROW, jnp.int32)])
    dst_p = dst_p.reshape(NW, NCHUNKS, CHUNK)

    h = x
    for l in range(N_LAYERS):
        parts = _sc_aggregate(h, src_p, dst_p)
        h = _mlp_call(h, parts,
                      W1[l], b1[l].reshape(1, HID),
                      W2[l], b2[l].reshape(1, HID),
                      gamma[l].reshape(1, HID), beta[l].reshape(1, HID))
    return (h, batch)


# speedup vs baseline: 3.1652x; 1.0860x over previous
import functools

import jax
import jax.numpy as jnp
from jax import lax
from jax.experimental import pallas as pl
from jax.experimental.pallas import tpu as pltpu
from jax.experimental.pallas import tpu_sc as plsc

N_NODES = 10000
N_EDGES = 320000
HID = 128
N_LAYERS = 3

NC = 2
NS = 16
NW = NC * NS

CHUNK = 128
NCHUNKS = 80
EPT = CHUNK * NCHUNKS
E_PAD = NW * EPT
N_HALVES = 2
HALF_CHUNKS = NCHUNKS // N_HALVES
HALF_EDGES = EPT // N_HALVES

DUMMY_ROW = N_NODES
AGG_ROWS = 10112
ZROWS_PER_TILE = AGG_ROWS // NS
OUT_ROWS_PER_TILE = 624

_sc_mesh = plsc.VectorSubcoreMesh(core_axis_name="c", subcore_axis_name="s")


@functools.partial(
    pl.kernel,
    out_type=jax.ShapeDtypeStruct((NC, N_NODES, HID), jnp.float32),
    mesh=_sc_mesh,
    scratch_types=[
        pltpu.VMEM((HALF_EDGES,), jnp.int32),
        pltpu.VMEM((HALF_CHUNKS, CHUNK), jnp.int32),
        pltpu.VMEM((CHUNK, HID), jnp.float32),
        pltpu.VMEM((CHUNK, HID), jnp.float32),
        pltpu.VMEM_SHARED((AGG_ROWS, HID), jnp.float32),
        pltpu.SemaphoreType.DMA,
        pltpu.SemaphoreType.DMA,
        pltpu.SemaphoreType.DMA,
        pltpu.SemaphoreType.DMA,
    ],
)
def _sc_aggregate(h_hbm, src_hbm, dst_hbm, out_hbm,
                  src_v, dst_v, b0, b1, agg_sh, g0, g1, s0, s1):
    c = lax.axis_index("c")
    s = lax.axis_index("s")
    wid = c * NS + s

    zeros16 = jnp.zeros((16,), jnp.float32)

    @pl.loop(0, CHUNK)
    def _(r):
        for cc in range(HID // 16):
            b0[r, pl.ds(cc * 16, 16)] = zeros16

    zbase = pl.multiple_of(s * ZROWS_PER_TILE, 8)
    for k in range(ZROWS_PER_TILE // CHUNK):
        pltpu.sync_copy(b0, agg_sh.at[pl.ds(zbase + k * CHUNK, CHUNK)])
    rem = ZROWS_PER_TILE % CHUNK
    if rem:
        pltpu.sync_copy(b0.at[pl.ds(0, rem)],
                        agg_sh.at[pl.ds(zbase + ZROWS_PER_TILE - rem, rem)])

    plsc.subcore_barrier()

    def gather_issue(cidx, buf, sem):
        pltpu.async_copy(h_hbm.at[src_v.at[pl.ds(cidx * CHUNK, CHUNK)]],
                         buf, sem)

    def gather_wait(buf, sem):
        pltpu.make_async_copy(h_hbm.at[src_v.at[pl.ds(0, CHUNK)]],
                              buf, sem).wait()

    def scatter_issue(cidx, buf, sem):
        pltpu.async_copy(buf, agg_sh.at[dst_v.at[cidx]], sem, add=True)

    def scatter_wait(buf, sem):
        pltpu.make_async_copy(buf, agg_sh.at[dst_v.at[0]], sem).wait()

    for half in range(N_HALVES):
        pltpu.sync_copy(
            src_hbm.at[pl.ds(wid * EPT + half * HALF_EDGES, HALF_EDGES)],
            src_v)
        pltpu.sync_copy(dst_hbm.at[wid * N_HALVES + half], dst_v)
        gather_issue(0, b0, g0)

        @pl.loop(0, HALF_CHUNKS // 2)
        def _(i):
            gather_wait(b0, g0)

            @pl.when(i > 0)
            def _():
                scatter_wait(b1, s1)

            gather_issue(2 * i + 1, b1, g1)
            scatter_issue(2 * i, b0, s0)

            gather_wait(b1, g1)

            @pl.when(i < HALF_CHUNKS // 2 - 1)
            def _():
                scatter_wait(b0, s0)
                gather_issue(2 * i + 2, b0, g0)

            scatter_issue(2 * i + 1, b1, s1)

        scatter_wait(b0, s0)
        scatter_wait(b1, s1)

    plsc.subcore_barrier()

    obase = pl.multiple_of(s * OUT_ROWS_PER_TILE, 8)

    @pl.when(s < NS - 1)
    def _():
        pltpu.sync_copy(agg_sh.at[pl.ds(obase, OUT_ROWS_PER_TILE)],
                        out_hbm.at[c].at[pl.ds(obase, OUT_ROWS_PER_TILE)])

    last_base = (NS - 1) * OUT_ROWS_PER_TILE
    last_rows = N_NODES - last_base

    @pl.when(s == NS - 1)
    def _():
        pltpu.sync_copy(agg_sh.at[pl.ds(last_base, last_rows)],
                        out_hbm.at[c].at[pl.ds(last_base, last_rows)])


def _mlp_body(h_ref, p_ref, w1_ref, b1_ref, w2_ref, b2_ref, g_ref, be_ref,
              o_ref):
    z = h_ref[...] + p_ref[0] + p_ref[1]
    z = jnp.dot(z, w1_ref[...], preferred_element_type=jnp.float32) + b1_ref[...]
    z = jnp.maximum(z, 0.0)
    z = jnp.dot(z, w2_ref[...], preferred_element_type=jnp.float32) + b2_ref[...]
    z = jnp.maximum(z, 0.0)
    mu = jnp.mean(z, axis=0, keepdims=True)
    zc = z - mu
    var = jnp.mean(zc * zc, axis=0, keepdims=True)
    o_ref[...] = zc * lax.rsqrt(var + 1e-5) * g_ref[...] + be_ref[...]


_mlp_call = pl.pallas_call(
    _mlp_body,
    out_shape=jax.ShapeDtypeStruct((N_NODES, HID), jnp.float32),
)


def kernel(x, edge_index, batch, W1, b1, W2, b2, gamma, beta):
    src = edge_index[0].astype(jnp.int32)
    dst = edge_index[1].astype(jnp.int32)
    pad = E_PAD - N_EDGES
    src_p = jnp.concatenate([src, jnp.zeros((pad,), jnp.int32)])
    dst_p = jnp.concatenate([dst, jnp.full((pad,), DUMMY_ROW, jnp.int32)])
    dst_p = dst_p.reshape(NW * N_HALVES, HALF_CHUNKS, CHUNK)

    h = x
    for l in range(N_LAYERS):
        parts = _sc_aggregate(h, src_p, dst_p)
        h = _mlp_call(h, parts,
                      W1[l], b1[l].reshape(1, HID),
                      W2[l], b2[l].reshape(1, HID),
                      gamma[l].reshape(1, HID), beta[l].reshape(1, HID))
    return (h, batch)


# speedup vs baseline: 9.9167x; 3.1330x over previous
import functools

import jax
import jax.numpy as jnp
from jax import lax
from jax.experimental import pallas as pl
from jax.experimental.pallas import tpu as pltpu
from jax.experimental.pallas import tpu_sc as plsc

N_NODES = 10000
N_EDGES = 320000
HID = 128
N_LAYERS = 3

NC = 2
NS = 16
NW = NC * NS

CHUNK = 128
NCHUNKS = 80
EPT = CHUNK * NCHUNKS
E_PAD = NW * EPT
N_HALVES = 2
HALF_CHUNKS = NCHUNKS // N_HALVES
HALF_EDGES = EPT // N_HALVES

DUMMY_ROW = N_NODES
AGG_ROWS = 10112
ZROWS_PER_TILE = AGG_ROWS // NS
OUT_ROWS_PER_TILE = 624

_sc_mesh = plsc.VectorSubcoreMesh(core_axis_name="c", subcore_axis_name="s")


@functools.partial(
    pl.kernel,
    out_type=jax.ShapeDtypeStruct((NC, N_NODES, HID), jnp.float32),
    mesh=_sc_mesh,
    scratch_types=[
        pltpu.VMEM((HALF_EDGES,), jnp.int32),
        pltpu.VMEM((HALF_CHUNKS, CHUNK), jnp.int32),
        pltpu.VMEM((CHUNK, HID), jnp.float32),
        pltpu.VMEM((CHUNK, HID), jnp.float32),
        pltpu.VMEM_SHARED((AGG_ROWS, HID), jnp.float32),
        pltpu.SemaphoreType.DMA,
        pltpu.SemaphoreType.DMA,
        pltpu.SemaphoreType.DMA,
        pltpu.SemaphoreType.DMA,
    ],
)
def _sc_aggregate(h_hbm, src_hbm, dst_hbm, out_hbm,
                  src_v, dst_v, b0, b1, agg_sh, g0, g1, s0, s1):
    c = lax.axis_index("c")
    s = lax.axis_index("s")
    wid = c * NS + s

    zeros16 = jnp.zeros((16,), jnp.float32)

    @pl.loop(0, CHUNK)
    def _(r):
        for cc in range(HID // 16):
            b0[r, pl.ds(cc * 16, 16)] = zeros16

    zbase = pl.multiple_of(s * ZROWS_PER_TILE, 8)
    for k in range(ZROWS_PER_TILE // CHUNK):
        pltpu.sync_copy(b0, agg_sh.at[pl.ds(zbase + k * CHUNK, CHUNK)])
    rem = ZROWS_PER_TILE % CHUNK
    if rem:
        pltpu.sync_copy(b0.at[pl.ds(0, rem)],
                        agg_sh.at[pl.ds(zbase + ZROWS_PER_TILE - rem, rem)])

    plsc.subcore_barrier()

    def gather_issue(cidx, buf, sem):
        pltpu.async_copy(h_hbm.at[src_v.at[pl.ds(cidx * CHUNK, CHUNK)]],
                         buf, sem)

    def gather_wait(buf, sem):
        pltpu.make_async_copy(h_hbm.at[src_v.at[pl.ds(0, CHUNK)]],
                              buf, sem).wait()

    def scatter_issue(cidx, buf, sem):
        pltpu.async_copy(buf, agg_sh.at[dst_v.at[cidx]], sem, add=True)

    def scatter_wait(buf, sem):
        pltpu.make_async_copy(buf, agg_sh.at[dst_v.at[0]], sem).wait()

    for half in range(N_HALVES):
        pltpu.sync_copy(
            src_hbm.at[pl.ds(wid * EPT + half * HALF_EDGES, HALF_EDGES)],
            src_v)
        pltpu.sync_copy(dst_hbm.at[wid * N_HALVES + half], dst_v)
        gather_issue(0, b0, g0)

        @pl.loop(0, HALF_CHUNKS // 2)
        def _(i):
            gather_wait(b0, g0)

            @pl.when(i > 0)
            def _():
                scatter_wait(b1, s1)

            gather_issue(2 * i + 1, b1, g1)
            scatter_issue(2 * i, b0, s0)

            gather_wait(b1, g1)

            @pl.when(i < HALF_CHUNKS // 2 - 1)
            def _():
                scatter_wait(b0, s0)
                gather_issue(2 * i + 2, b0, g0)

            scatter_issue(2 * i + 1, b1, s1)

        scatter_wait(b0, s0)
        scatter_wait(b1, s1)

    plsc.subcore_barrier()

    obase = pl.multiple_of(s * OUT_ROWS_PER_TILE, 8)

    @pl.when(s < NS - 1)
    def _():
        pltpu.sync_copy(agg_sh.at[pl.ds(obase, OUT_ROWS_PER_TILE)],
                        out_hbm.at[c].at[pl.ds(obase, OUT_ROWS_PER_TILE)])

    last_base = (NS - 1) * OUT_ROWS_PER_TILE
    last_rows = N_NODES - last_base

    @pl.when(s == NS - 1)
    def _():
        pltpu.sync_copy(agg_sh.at[pl.ds(last_base, last_rows)],
                        out_hbm.at[c].at[pl.ds(last_base, last_rows)])


def _mlp_body(h_ref, p_ref, w1_ref, b1_ref, w2_ref, b2_ref, g_ref, be_ref,
              o_ref):
    z = h_ref[...] + p_ref[0] + p_ref[1]
    z = jnp.dot(z, w1_ref[...], preferred_element_type=jnp.float32) + b1_ref[...]
    z = jnp.maximum(z, 0.0)
    z = jnp.dot(z, w2_ref[...], preferred_element_type=jnp.float32) + b2_ref[...]
    z = jnp.maximum(z, 0.0)
    mu = jnp.mean(z, axis=0, keepdims=True)
    zc = z - mu
    var = jnp.mean(zc * zc, axis=0, keepdims=True)
    o_ref[...] = zc * lax.rsqrt(var + 1e-5) * g_ref[...] + be_ref[...]


_mlp_call = pl.pallas_call(
    _mlp_body,
    out_shape=jax.ShapeDtypeStruct((N_NODES, HID), jnp.float32),
)


def kernel(x, edge_index, batch, W1, b1, W2, b2, gamma, beta):
    src = edge_index[0].astype(jnp.int32)
    dst = edge_index[1].astype(jnp.int32)
    pad = E_PAD - N_EDGES
    pad_src = (jnp.arange(pad, dtype=jnp.int32) * 131) % N_NODES
    pad_dst = DUMMY_ROW + (jnp.arange(pad, dtype=jnp.int32) % (AGG_ROWS - N_NODES))
    src_p = jnp.concatenate([src, pad_src])
    dst_p = jnp.concatenate([dst, pad_dst])
    dst_p = dst_p.reshape(NW * N_HALVES, HALF_CHUNKS, CHUNK)

    h = x
    for l in range(N_LAYERS):
        parts = _sc_aggregate(h, src_p, dst_p)
        h = _mlp_call(h, parts,
                      W1[l], b1[l].reshape(1, HID),
                      W2[l], b2[l].reshape(1, HID),
                      gamma[l].reshape(1, HID), beta[l].reshape(1, HID))
    return (h, batch)


# speedup vs baseline: 10.5028x; 1.0591x over previous
import functools

import jax
import jax.numpy as jnp
from jax import lax
from jax.experimental import pallas as pl
from jax.experimental.pallas import tpu as pltpu
from jax.experimental.pallas import tpu_sc as plsc

N_NODES = 10000
N_EDGES = 320000
HID = 128
N_LAYERS = 3

NC = 2
NS = 16
NW = NC * NS

CHUNK = 80
NCHUNKS = 128
EPT = CHUNK * NCHUNKS
E_PAD = NW * EPT
NBUF = 4
STAGES = 4
STAGE_CHUNKS = NCHUNKS // STAGES
STAGE_EDGES = EPT // STAGES

DUMMY_ROW = N_NODES
AGG_ROWS = 10112
ZROWS_PER_TILE = AGG_ROWS // NS
OUT_ROWS_PER_TILE = 624

_sc_mesh = plsc.VectorSubcoreMesh(core_axis_name="c", subcore_axis_name="s")


@functools.partial(
    pl.kernel,
    out_type=jax.ShapeDtypeStruct((NC, N_NODES, HID), jnp.float32),
    mesh=_sc_mesh,
    scratch_types=[
        pltpu.VMEM((STAGE_EDGES,), jnp.int32),
        pltpu.VMEM((STAGE_CHUNKS, CHUNK), jnp.int32),
        pltpu.VMEM((CHUNK, HID), jnp.float32),
        pltpu.VMEM((CHUNK, HID), jnp.float32),
        pltpu.VMEM((CHUNK, HID), jnp.float32),
        pltpu.VMEM((CHUNK, HID), jnp.float32),
        pltpu.VMEM_SHARED((AGG_ROWS, HID), jnp.float32),
        pltpu.SemaphoreType.DMA,
        pltpu.SemaphoreType.DMA,
        pltpu.SemaphoreType.DMA,
        pltpu.SemaphoreType.DMA,
        pltpu.SemaphoreType.DMA,
        pltpu.SemaphoreType.DMA,
        pltpu.SemaphoreType.DMA,
        pltpu.SemaphoreType.DMA,
    ],
)
def _sc_aggregate(h_hbm, src_hbm, dst_hbm, out_hbm,
                  src_v, dst_v, b0, b1, b2, b3, agg_sh,
                  g0, g1, g2, g3, s0, s1, s2, s3):
    c = lax.axis_index("c")
    s = lax.axis_index("s")
    wid = c * NS + s
    bufs = [b0, b1, b2, b3]
    gsems = [g0, g1, g2, g3]
    ssems = [s0, s1, s2, s3]

    zeros16 = jnp.zeros((16,), jnp.float32)

    @pl.loop(0, CHUNK)
    def _(r):
        for cc in range(HID // 16):
            b0[r, pl.ds(cc * 16, 16)] = zeros16

    zbase = pl.multiple_of(s * ZROWS_PER_TILE, 8)
    for k in range(ZROWS_PER_TILE // CHUNK):
        pltpu.sync_copy(b0, agg_sh.at[pl.ds(zbase + k * CHUNK, CHUNK)])
    rem = ZROWS_PER_TILE % CHUNK
    if rem:
        pltpu.sync_copy(b0.at[pl.ds(0, rem)],
                        agg_sh.at[pl.ds(zbase + ZROWS_PER_TILE - rem, rem)])

    plsc.subcore_barrier()

    def gather_issue(cidx, b):
        pltpu.async_copy(h_hbm.at[src_v.at[pl.ds(cidx * CHUNK, CHUNK)]],
                         bufs[b], gsems[b])

    def gather_wait(b):
        pltpu.make_async_copy(h_hbm.at[src_v.at[pl.ds(0, CHUNK)]],
                              bufs[b], gsems[b]).wait()

    def scatter_issue(cidx, b):
        pltpu.async_copy(bufs[b], agg_sh.at[dst_v.at[cidx]], ssems[b],
                         add=True)

    def scatter_wait(b):
        pltpu.make_async_copy(bufs[b], agg_sh.at[dst_v.at[0]], ssems[b]).wait()

    for stage in range(STAGES):
        pltpu.sync_copy(
            src_hbm.at[pl.ds(wid * EPT + stage * STAGE_EDGES, STAGE_EDGES)],
            src_v)
        pltpu.sync_copy(dst_hbm.at[wid * STAGES + stage], dst_v)
        gather_issue(0, 0)
        gather_issue(1, 1)

        @pl.loop(0, STAGE_CHUNKS // NBUF)
        def _(i):
            base = i * NBUF
            for k in range(NBUF):
                gather_wait(k)
                scatter_issue(base + k, k)
                nb = (k + 2) % NBUF
                if k < 2:
                    @pl.when(i > 0)
                    def _():
                        scatter_wait(nb)
                    gather_issue(base + k + 2, nb)
                else:
                    scatter_wait(nb)

                    @pl.when(i < STAGE_CHUNKS // NBUF - 1)
                    def _():
                        gather_issue(base + k + 2, nb)

        scatter_wait(2)
        scatter_wait(3)

    plsc.subcore_barrier()

    obase = pl.multiple_of(s * OUT_ROWS_PER_TILE, 8)

    @pl.when(s < NS - 1)
    def _():
        pltpu.sync_copy(agg_sh.at[pl.ds(obase, OUT_ROWS_PER_TILE)],
                        out_hbm.at[c].at[pl.ds(obase, OUT_ROWS_PER_TILE)])

    last_base = (NS - 1) * OUT_ROWS_PER_TILE
    last_rows = N_NODES - last_base

    @pl.when(s == NS - 1)
    def _():
        pltpu.sync_copy(agg_sh.at[pl.ds(last_base, last_rows)],
                        out_hbm.at[c].at[pl.ds(last_base, last_rows)])


def _mlp_body(h_ref, p_ref, w1_ref, b1_ref, w2_ref, b2_ref, g_ref, be_ref,
              o_ref):
    z = h_ref[...] + p_ref[0] + p_ref[1]
    z = jnp.dot(z, w1_ref[...], preferred_element_type=jnp.float32) + b1_ref[...]
    z = jnp.maximum(z, 0.0)
    z = jnp.dot(z, w2_ref[...], preferred_element_type=jnp.float32) + b2_ref[...]
    z = jnp.maximum(z, 0.0)
    mu = jnp.mean(z, axis=0, keepdims=True)
    zc = z - mu
    var = jnp.mean(zc * zc, axis=0, keepdims=True)
    o_ref[...] = zc * lax.rsqrt(var + 1e-5) * g_ref[...] + be_ref[...]


_mlp_call = pl.pallas_call(
    _mlp_body,
    out_shape=jax.ShapeDtypeStruct((N_NODES, HID), jnp.float32),
)


def kernel(x, edge_index, batch, W1, b1, W2, b2, gamma, beta):
    src = edge_index[0].astype(jnp.int32)
    dst = edge_index[1].astype(jnp.int32)
    pad = E_PAD - N_EDGES
    pad_src = (jnp.arange(pad, dtype=jnp.int32) * 131) % N_NODES
    pad_dst = DUMMY_ROW + (jnp.arange(pad, dtype=jnp.int32) % (AGG_ROWS - N_NODES))
    src_p = jnp.concatenate([src, pad_src])
    dst_p = jnp.concatenate([dst, pad_dst])
    dst_p = dst_p.reshape(NW * STAGES, STAGE_CHUNKS, CHUNK)

    h = x
    for l in range(N_LAYERS):
        parts = _sc_aggregate(h, src_p, dst_p)
        h = _mlp_call(h, parts,
                      W1[l], b1[l].reshape(1, HID),
                      W2[l], b2[l].reshape(1, HID),
                      gamma[l].reshape(1, HID), beta[l].reshape(1, HID))
    return (h, batch)
